# baseline (speedup 1.0000x reference)
import numpy as np
import jax
import jax.numpy as jnp
from jax.experimental import pallas as pl

H = 1024
HQ = 16
HK = 2
D = 64
KS = 32
KSTR = 16
BS = 64
TOPK = 8
INIT_B = 1
LOCAL_B = 2
WIN = 256
THETA = 10000.0
B = 4
L = 1024
N = B * L
G = HQ // HK
C = (L - KS) // KSTR + 1
CP = 64
NB = L // BS
SCALE = 1.0 / float(np.sqrt(D))
HALF = D // 2

RB = 512
QB = 256
NQB = L // QB

F32 = jnp.float32


def _rope_tables(pos):
    inv = (1.0 / (THETA ** (np.arange(HALF, dtype=np.float32) / HALF))).astype(np.float32)
    ang = (pos.astype(np.float32)[:, None] * inv[None, :]).astype(np.float32)
    return (np.cos(ang.astype(np.float64)).astype(np.float32),
            np.sin(ang.astype(np.float64)).astype(np.float32))


COS_Q, SIN_Q = _rope_tables(np.arange(L))
COS_C, SIN_C = _rope_tables(np.arange(CP) * KSTR)

_i = np.arange(L)[:, None]
_c = np.arange(CP)[None, :]
CMASK = (((_i >= (_c * KSTR + KS - 1)) & (_c < C)).astype(np.float32))

_jstart = np.arange(CP) * KSTR
_bstart = np.arange(NB) * BS
OVERLAP = (((_jstart[:, None] < _bstart[None, :] + BS)
            & (_jstart[:, None] + KS > _bstart[None, :]))
           & (np.arange(CP)[:, None] < C)).astype(np.float32)

_qblk = np.arange(L) // BS
_bidx = np.arange(NB)
FORCED = ((_bidx[None, :] < INIT_B)
          | ((_bidx[None, :] > _qblk[:, None] - LOCAL_B)
             & (_bidx[None, :] <= _qblk[:, None]))).astype(np.float32)
FUTURE = (_bidx[None, :] > _qblk[:, None]).astype(np.float32)

EXPAND = (np.arange(L)[None, :] // BS == _bidx[:, None]).astype(np.float32)


def _rope(xb, cos, sin):
    x1 = xb[:, :HALF]
    x2 = xb[:, HALF:]
    return jnp.concatenate([x1 * cos - x2 * sin, x2 * cos + x1 * sin], axis=-1)


def _masked_softmax(s, m):
    sm = jnp.where(m, s, -1e30)
    mx = jnp.max(sm, axis=-1, keepdims=True)
    e = jnp.where(m, jnp.exp(sm - mx), 0.0)
    return e / jnp.maximum(jnp.sum(e, axis=-1, keepdims=True), 1e-20)


def _proj_kernel(x_ref, wq_ref, wk_ref, wv_ref, wg_ref,
                 q_ref, k_ref, v_ref, gate_ref):
    xf = x_ref[...]
    q_ref[0] = jnp.dot(xf, wq_ref[...], preferred_element_type=F32)
    kk = jnp.dot(xf, wk_ref[...], preferred_element_type=F32)
    vv = jnp.dot(xf, wv_ref[...], preferred_element_type=F32)
    for h in range(HK):
        k_ref[0, h] = kk[:, h * D:(h + 1) * D]
        v_ref[0, h] = vv[:, h * D:(h + 1) * D]
    gate_ref[0] = jax.nn.sigmoid(jnp.dot(xf, wg_ref[...], preferred_element_type=F32))


def _compress_kernel(k_ref, v_ref, wck_ref, wcv_ref, pe_ref, cosc_ref, sinc_ref,
                     ck_ref, cv_ref):
    k3 = k_ref[0, 0].reshape(L // KSTR, KSTR, D)
    v3 = v_ref[0, 0].reshape(L // KSTR, KSTR, D)
    acc_k = jnp.zeros((C, D), F32)
    acc_v = jnp.zeros((C, D), F32)
    for t in range(KS):
        off, tt = divmod(t, KSTR)
        xk = k3[off:off + C, tt, :] + pe_ref[0, t]
        xv = v3[off:off + C, tt, :]
        acc_k = acc_k + jnp.dot(xk, wck_ref[0, t * D:(t + 1) * D, :],
                                preferred_element_type=F32)
        acc_v = acc_v + jnp.dot(xv, wcv_ref[0, t * D:(t + 1) * D, :],
                                preferred_element_type=F32)
    ck = _rope(acc_k, cosc_ref[:C], sinc_ref[:C])
    pad = jnp.zeros((CP - C, D), F32)
    ck_ref[0, 0] = jnp.concatenate([ck, pad], axis=0)
    cv_ref[0, 0] = jnp.concatenate([acc_v, pad], axis=0)


def _select_kernel(q_ref, ck_ref, cv_ref, cosq_ref, sinq_ref, cmask_ref,
                   ov_ref, forced_ref, future_ref,
                   comp_ref, bm_ref):
    ck = ck_ref[0, 0]
    cv = cv_ref[0, 0]
    cmask = cmask_ref[...] > 0.5
    cos = cosq_ref[...]
    sin = sinq_ref[...]
    hscore = jnp.zeros((L, CP), F32)
    for g in range(G):
        qr = _rope(q_ref[0][:, g * D:(g + 1) * D], cos, sin)
        s = jax.lax.dot_general(qr, ck, (((1,), (1,)), ((), ())),
                                preferred_element_type=F32) * SCALE
        p = _masked_softmax(s, cmask)
        comp_ref[0, 0, g] = jnp.dot(p, cv, preferred_element_type=F32)
        hscore = hscore + p
    bscore = jnp.dot(hscore, ov_ref[...], preferred_element_type=F32)
    bscore = jnp.where(forced_ref[...] > 0.5, 1e9, bscore)
    bscore = jnp.where(future_ref[...] > 0.5, -1e30, bscore)
    iota = jax.lax.broadcasted_iota(jnp.int32, (L, NB), 1)
    mask = jnp.zeros((L, NB), F32)
    bs = bscore
    for _ in range(TOPK):
        mx = jnp.max(bs, axis=-1, keepdims=True)
        first = jnp.min(jnp.where(bs == mx, iota, NB), axis=-1, keepdims=True)
        sel = iota == first
        mask = jnp.where(sel & (mx > -1e29), 1.0, mask)
        bs = jnp.where(sel, -3e38, bs)
    bm_ref[0, 0] = mask


def _attn_kernel(q_ref, k_ref, v_ref, comp_ref, bm_ref, gate_ref,
                 cosq_ref, sinq_ref, expand_ref, out_ref):
    qb = pl.program_id(2)
    k = _rope(k_ref[0, 0], cosq_ref[...], sinq_ref[...]).astype(jnp.bfloat16)
    v = v_ref[0, 0].astype(jnp.bfloat16)
    riota = jax.lax.broadcasted_iota(jnp.int32, (QB, L), 0) + qb * QB
    tiota = jax.lax.broadcasted_iota(jnp.int32, (QB, L), 1)
    causal = riota >= tiota
    wmask = causal & ((riota - tiota) <= WIN)
    km = jnp.dot(bm_ref[0, 0], expand_ref[...], preferred_element_type=F32)
    smask = causal & (km > 0.5)
    cos = cosq_ref[pl.ds(qb * QB, QB), :]
    sin = sinq_ref[pl.ds(qb * QB, QB), :]
    gate = gate_ref[0]
    g0 = gate[:, 0:1]
    g1 = gate[:, 1:2]
    g2 = gate[:, 2:3]
    for g in range(G):
        qr = _rope(q_ref[0][:, g * D:(g + 1) * D], cos, sin).astype(jnp.bfloat16)
        s = jax.lax.dot_general(qr, k, (((1,), (1,)), ((), ())),
                                preferred_element_type=F32) * SCALE
        ps = _masked_softmax(s, smask).astype(jnp.bfloat16)
        sp_out = jnp.dot(ps, v, preferred_element_type=F32)
        pw = _masked_softmax(s, wmask).astype(jnp.bfloat16)
        sl_out = jnp.dot(pw, v, preferred_element_type=F32)
        comb = g0 * comp_ref[0, 0, g] + g1 * sp_out + g2 * sl_out
        out_ref[0, :, g * D:(g + 1) * D] = comb


def _outproj_kernel(a_ref, wo_ref, o_ref):
    o_ref[...] = jnp.dot(a_ref[...].astype(jnp.bfloat16), wo_ref[...],
                         preferred_element_type=F32)


def _full(shape):
    nd = len(shape)
    return pl.BlockSpec(shape, lambda *_: (0,) * nd)


def kernel(x, cu_seqlens, Wq, Wk, Wv, Wo, Wck, Wcv, pe, Wg):
    del cu_seqlens
    bf16 = jnp.bfloat16
    Wq = Wq.astype(bf16)
    Wk = Wk.astype(bf16)
    Wv = Wv.astype(bf16)
    Wo = Wo.astype(bf16)
    Wg_pad = jnp.concatenate([Wg, jnp.zeros((H, 5), Wg.dtype)],
                             axis=1).astype(bf16)
    nrb = L // RB

    q, k, v, gate = pl.pallas_call(
        _proj_kernel,
        grid=(B * nrb,),
        in_specs=[
            pl.BlockSpec((RB, H), lambda i: (i, 0)),
            _full((H, HQ * D)),
            _full((H, HK * D)),
            _full((H, HK * D)),
            _full((H, 8)),
        ],
        out_specs=[
            pl.BlockSpec((1, RB, HQ * D), lambda i: (i // nrb, i % nrb, 0)),
            pl.BlockSpec((1, HK, RB, D), lambda i: (i // nrb, 0, i % nrb, 0)),
            pl.BlockSpec((1, HK, RB, D), lambda i: (i // nrb, 0, i % nrb, 0)),
            pl.BlockSpec((1, RB, 8), lambda i: (i // nrb, i % nrb, 0)),
        ],
        out_shape=[
            jax.ShapeDtypeStruct((B, L, HQ * D), F32),
            jax.ShapeDtypeStruct((B, HK, L, D), F32),
            jax.ShapeDtypeStruct((B, HK, L, D), F32),
            jax.ShapeDtypeStruct((B, L, 8), F32),
        ],
    )(x, Wq, Wk, Wv, Wg_pad)

    ck, cv = pl.pallas_call(
        _compress_kernel,
        grid=(B, HK),
        in_specs=[
            pl.BlockSpec((1, 1, L, D), lambda b, h: (b, h, 0, 0)),
            pl.BlockSpec((1, 1, L, D), lambda b, h: (b, h, 0, 0)),
            pl.BlockSpec((1, KS * D, D), lambda b, h: (h, 0, 0)),
            pl.BlockSpec((1, KS * D, D), lambda b, h: (h, 0, 0)),
            pl.BlockSpec((1, KS, D), lambda b, h: (h, 0, 0)),
            _full((CP, HALF)),
            _full((CP, HALF)),
        ],
        out_specs=[
            pl.BlockSpec((1, 1, CP, D), lambda b, h: (b, h, 0, 0)),
            pl.BlockSpec((1, 1, CP, D), lambda b, h: (b, h, 0, 0)),
        ],
        out_shape=[
            jax.ShapeDtypeStruct((B, HK, CP, D), F32),
            jax.ShapeDtypeStruct((B, HK, CP, D), F32),
        ],
    )(k, v, Wck, Wcv, pe, COS_C, SIN_C)

    comp, bm = pl.pallas_call(
        _select_kernel,
        grid=(B, HK),
        in_specs=[
            pl.BlockSpec((1, L, G * D), lambda b, h: (b, 0, h)),
            pl.BlockSpec((1, 1, CP, D), lambda b, h: (b, h, 0, 0)),
            pl.BlockSpec((1, 1, CP, D), lambda b, h: (b, h, 0, 0)),
            _full((L, HALF)),
            _full((L, HALF)),
            _full((L, CP)),
            _full((CP, NB)),
            _full((L, NB)),
            _full((L, NB)),
        ],
        out_specs=[
            pl.BlockSpec((1, 1, G, L, D), lambda b, h: (b, h, 0, 0, 0)),
            pl.BlockSpec((1, 1, L, NB), lambda b, h: (b, h, 0, 0)),
        ],
        out_shape=[
            jax.ShapeDtypeStruct((B, HK, G, L, D), F32),
            jax.ShapeDtypeStruct((B, HK, L, NB), F32),
        ],
    )(q, ck, cv, COS_Q, SIN_Q, CMASK, OVERLAP, FORCED, FUTURE)

    att = pl.pallas_call(
        _attn_kernel,
        grid=(B, HK, NQB),
        in_specs=[
            pl.BlockSpec((1, QB, G * D), lambda b, h, i: (b, i, h)),
            pl.BlockSpec((1, 1, L, D), lambda b, h, i: (b, h, 0, 0)),
            pl.BlockSpec((1, 1, L, D), lambda b, h, i: (b, h, 0, 0)),
            pl.BlockSpec((1, 1, G, QB, D), lambda b, h, i: (b, h, 0, i, 0)),
            pl.BlockSpec((1, 1, QB, NB), lambda b, h, i: (b, h, i, 0)),
            pl.BlockSpec((1, QB, 8), lambda b, h, i: (b, i, 0)),
            _full((L, HALF)),
            _full((L, HALF)),
            _full((NB, L)),
        ],
        out_specs=pl.BlockSpec((1, QB, G * D), lambda b, h, i: (b, i, h)),
        out_shape=jax.ShapeDtypeStruct((B, L, HQ * D), F32),
    )(q, k, v, comp, bm, gate, COS_Q, SIN_Q, EXPAND)

    out = pl.pallas_call(
        _outproj_kernel,
        grid=(N // RB,),
        in_specs=[
            pl.BlockSpec((RB, HQ * D), lambda i: (i, 0)),
            _full((HQ * D, H)),
        ],
        out_specs=pl.BlockSpec((RB, H), lambda i: (i, 0)),
        out_shape=jax.ShapeDtypeStruct((N, H), F32),
    )(att.reshape(N, HQ * D), Wo)

    return out


# speedup vs baseline: 1.5794x; 1.5794x over previous
import numpy as np
import jax
import jax.numpy as jnp
from jax.experimental import pallas as pl

H = 1024
HQ = 16
HK = 2
D = 64
KS = 32
KSTR = 16
BS = 64
TOPK = 8
INIT_B = 1
LOCAL_B = 2
WIN = 256
THETA = 10000.0
B = 4
L = 1024
N = B * L
G = HQ // HK
C = (L - KS) // KSTR + 1
CP = 64
NB = L // BS
SCALE = 1.0 / float(np.sqrt(D))
HALF = D // 2

RB = 512
QB = 256
NQB = L // QB

F32 = jnp.float32
BF16 = jnp.bfloat16


def _rope_tables(pos):
    inv = (1.0 / (THETA ** (np.arange(HALF, dtype=np.float32) / HALF))).astype(np.float32)
    ang = (pos.astype(np.float32)[:, None] * inv[None, :]).astype(np.float32)
    return (np.cos(ang.astype(np.float64)).astype(np.float32),
            np.sin(ang.astype(np.float64)).astype(np.float32))


COS_Q, SIN_Q = _rope_tables(np.arange(L))
COS_C, SIN_C = _rope_tables(np.arange(CP) * KSTR)

COSW_Q = np.tile(COS_Q, (1, 2 * HQ)).astype(np.float32)
SINW_Q = np.tile(SIN_Q, (1, 2 * HQ)).astype(np.float32)
COSW_K = np.tile(COS_Q, (1, 2 * HK)).astype(np.float32)
SINW_K = np.tile(SIN_Q, (1, 2 * HK)).astype(np.float32)

_i = np.arange(L)[:, None]
_c = np.arange(CP)[None, :]
CMASK = (((_i >= (_c * KSTR + KS - 1)) & (_c < C)).astype(np.float32))

_jstart = np.arange(CP) * KSTR
_bstart = np.arange(NB) * BS
OVERLAP = (((_jstart[:, None] < _bstart[None, :] + BS)
            & (_jstart[:, None] + KS > _bstart[None, :]))
           & (np.arange(CP)[:, None] < C)).astype(np.float32)

_qblk = np.arange(L) // BS
_bidx = np.arange(NB)
_forced = ((_bidx[None, :] < INIT_B)
           | ((_bidx[None, :] > _qblk[:, None] - LOCAL_B)
              & (_bidx[None, :] <= _qblk[:, None])))
_future = _bidx[None, :] > _qblk[:, None]
FORCEDV = np.where(_forced, 1e9 + 128.0 * _bidx[None, :], 0.0).astype(np.float32)
FORCED = _forced.astype(np.float32)
FUTURE = _future.astype(np.float32)

EXPAND = (np.arange(L)[None, :] // BS == _bidx[:, None]).astype(np.float32)


def _rope(xb, cos, sin):
    x1 = xb[:, :HALF]
    x2 = xb[:, HALF:]
    return jnp.concatenate([x1 * cos - x2 * sin, x2 * cos + x1 * sin], axis=-1)


def _proj_kernel(x_ref, wq_ref, wqr_ref, wk_ref, wkr_ref, wv_ref, wg_ref,
                 cq_ref, sq_ref, ckk_ref, skk_ref,
                 q_ref, kraw_ref, krot_ref, v_ref, gate_ref):
    xf = x_ref[...]
    qa = jnp.dot(xf, wq_ref[...], preferred_element_type=F32)
    qb = jnp.dot(xf, wqr_ref[...], preferred_element_type=F32)
    q_ref[0] = qa * cq_ref[...] + qb * sq_ref[...]
    ka = jnp.dot(xf, wk_ref[...], preferred_element_type=F32)
    kb = jnp.dot(xf, wkr_ref[...], preferred_element_type=F32)
    kr = ka * ckk_ref[...] + kb * skk_ref[...]
    vv = jnp.dot(xf, wv_ref[...], preferred_element_type=F32)
    for h in range(HK):
        kraw_ref[0, h] = ka[:, h * D:(h + 1) * D]
        krot_ref[0, h] = kr[:, h * D:(h + 1) * D]
        v_ref[0, h] = vv[:, h * D:(h + 1) * D]
    gate_ref[0] = jax.nn.sigmoid(jnp.dot(xf, wg_ref[...], preferred_element_type=F32))


def _compress_kernel(k_ref, v_ref, wck_ref, wcv_ref, pe_ref, cosc_ref, sinc_ref,
                     ck_ref, cv_ref):
    k3 = k_ref[0, 0].reshape(L // KSTR, KSTR, D)
    v3 = v_ref[0, 0].reshape(L // KSTR, KSTR, D)
    acc_k = jnp.zeros((C, D), F32)
    acc_v = jnp.zeros((C, D), F32)
    for t in range(KS):
        off, tt = divmod(t, KSTR)
        xk = k3[off:off + C, tt, :] + pe_ref[0, t]
        xv = v3[off:off + C, tt, :]
        acc_k = acc_k + jnp.dot(xk, wck_ref[0, t * D:(t + 1) * D, :],
                                preferred_element_type=F32)
        acc_v = acc_v + jnp.dot(xv, wcv_ref[0, t * D:(t + 1) * D, :],
                                preferred_element_type=F32)
    ck = _rope(acc_k, cosc_ref[:C], sinc_ref[:C])
    pad = jnp.zeros((CP - C, D), F32)
    ck_ref[0, 0] = jnp.concatenate([ck, pad], axis=0)
    cv_ref[0, 0] = jnp.concatenate([acc_v, pad], axis=0)


def _select_kernel(q_ref, ck_ref, cv_ref, cmask_ref, ov_ref,
                   forcedv_ref, future_ref,
                   comp_ref, bm_ref):
    ck = ck_ref[0, 0]
    cv = cv_ref[0, 0]
    cmaskf = cmask_ref[...]
    cmask = cmaskf > 0.5
    hscore = jnp.zeros((L, CP), F32)
    for g in range(G):
        qg = q_ref[0][:, g * D:(g + 1) * D]
        s = jax.lax.dot_general(qg, ck, (((1,), (1,)), ((), ())),
                                preferred_element_type=F32) * SCALE
        sm = jnp.where(cmask, s, -1e30)
        mx = jnp.max(sm, axis=-1, keepdims=True)
        e = jnp.exp(sm - mx) * cmaskf
        recip = 1.0 / jnp.maximum(jnp.sum(e, axis=-1, keepdims=True), 1e-20)
        p = e * recip
        comp_ref[0, 0, g] = jnp.dot(p, cv, preferred_element_type=F32)
        hscore = hscore + p
    bscore = jnp.dot(hscore, ov_ref[...], preferred_element_type=F32)
    fv = forcedv_ref[...]
    bscore = jnp.where(fv > 0.5, fv, bscore)
    bscore = jnp.where(future_ref[...] > 0.5, -1e30, bscore)
    bs = bscore
    thr = None
    for _ in range(TOPK):
        thr = jnp.max(bs, axis=-1, keepdims=True)
        bs = jnp.where(bs >= thr, -3e38, bs)
    bm_ref[0, 0] = jnp.where((bscore >= thr) & (bscore > -1e29), 1.0, 0.0)


def _attn_kernel(q_ref, k_ref, v_ref, comp_ref, bm_ref, gate_ref, expand_ref,
                 out_ref):
    qb = pl.program_id(2)
    k = k_ref[0, 0]
    v = v_ref[0, 0]
    riota = jax.lax.broadcasted_iota(jnp.int32, (QB, L), 0) + qb * QB
    tiota = jax.lax.broadcasted_iota(jnp.int32, (QB, L), 1)
    causal = riota >= tiota
    wmaskf = jnp.where(causal & ((riota - tiota) <= WIN), 1.0, 0.0)
    km = jnp.dot(bm_ref[0, 0], expand_ref[...], preferred_element_type=F32)
    smaskf = jnp.where(causal, km, 0.0)
    union = (smaskf + wmaskf) > 0.5
    gate = gate_ref[0]
    g0 = gate[:, 0:1]
    g1 = gate[:, 1:2]
    g2 = gate[:, 2:3]
    for g in range(G):
        qg = q_ref[0][:, g * D:(g + 1) * D]
        s = jax.lax.dot_general(qg, k, (((1,), (1,)), ((), ())),
                                preferred_element_type=F32) * SCALE
        sm = jnp.where(union, s, -1e30)
        mx = jnp.max(sm, axis=-1, keepdims=True)
        eu = jnp.exp(sm - mx)
        es = eu * smaskf
        ew = eu * wmaskf
        rs = 1.0 / jnp.maximum(jnp.sum(es, axis=-1, keepdims=True), 1e-20)
        rw = 1.0 / jnp.maximum(jnp.sum(ew, axis=-1, keepdims=True), 1e-20)
        sp_out = jnp.dot(es, v, preferred_element_type=F32) * rs
        sl_out = jnp.dot(ew, v, preferred_element_type=F32) * rw
        comb = g0 * comp_ref[0, 0, g] + g1 * sp_out + g2 * sl_out
        out_ref[0, :, g * D:(g + 1) * D] = comb


def _outproj_kernel(a_ref, wo_ref, o_ref):
    o_ref[...] = jnp.dot(a_ref[...].astype(BF16), wo_ref[...],
                         preferred_element_type=F32)


def _full(shape):
    nd = len(shape)
    return pl.BlockSpec(shape, lambda *_: (0,) * nd)


def _rot_w(w, nh):
    w3 = w.reshape(H, nh, 2, HALF)
    return jnp.concatenate([-w3[:, :, 1:2], w3[:, :, 0:1]], axis=2).reshape(w.shape)


def kernel(x, cu_seqlens, Wq, Wk, Wv, Wo, Wck, Wcv, pe, Wg):
    del cu_seqlens
    Wq = Wq.astype(BF16)
    Wk = Wk.astype(BF16)
    Wv = Wv.astype(BF16)
    Wo = Wo.astype(BF16)
    Wqr = _rot_w(Wq, HQ)
    Wkr = _rot_w(Wk, HK)
    Wg_pad = jnp.concatenate([Wg, jnp.zeros((H, 5), Wg.dtype)],
                             axis=1).astype(BF16)
    nrb = L // RB

    q, kraw, krot, v, gate = pl.pallas_call(
        _proj_kernel,
        grid=(B * nrb,),
        in_specs=[
            pl.BlockSpec((RB, H), lambda i: (i, 0)),
            _full((H, HQ * D)),
            _full((H, HQ * D)),
            _full((H, HK * D)),
            _full((H, HK * D)),
            _full((H, HK * D)),
            _full((H, 8)),
            pl.BlockSpec((RB, HQ * D), lambda i: (i % nrb, 0)),
            pl.BlockSpec((RB, HQ * D), lambda i: (i % nrb, 0)),
            pl.BlockSpec((RB, HK * D), lambda i: (i % nrb, 0)),
            pl.BlockSpec((RB, HK * D), lambda i: (i % nrb, 0)),
        ],
        out_specs=[
            pl.BlockSpec((1, RB, HQ * D), lambda i: (i // nrb, i % nrb, 0)),
            pl.BlockSpec((1, HK, RB, D), lambda i: (i // nrb, 0, i % nrb, 0)),
            pl.BlockSpec((1, HK, RB, D), lambda i: (i // nrb, 0, i % nrb, 0)),
            pl.BlockSpec((1, HK, RB, D), lambda i: (i // nrb, 0, i % nrb, 0)),
            pl.BlockSpec((1, RB, 8), lambda i: (i // nrb, i % nrb, 0)),
        ],
        out_shape=[
            jax.ShapeDtypeStruct((B, L, HQ * D), F32),
            jax.ShapeDtypeStruct((B, HK, L, D), F32),
            jax.ShapeDtypeStruct((B, HK, L, D), F32),
            jax.ShapeDtypeStruct((B, HK, L, D), F32),
            jax.ShapeDtypeStruct((B, L, 8), F32),
        ],
    )(x, Wq, Wqr, Wk, Wkr, Wv, Wg_pad, COSW_Q, SINW_Q, COSW_K, SINW_K)

    ck, cv = pl.pallas_call(
        _compress_kernel,
        grid=(B, HK),
        in_specs=[
            pl.BlockSpec((1, 1, L, D), lambda b, h: (b, h, 0, 0)),
            pl.BlockSpec((1, 1, L, D), lambda b, h: (b, h, 0, 0)),
            pl.BlockSpec((1, KS * D, D), lambda b, h: (h, 0, 0)),
            pl.BlockSpec((1, KS * D, D), lambda b, h: (h, 0, 0)),
            pl.BlockSpec((1, KS, D), lambda b, h: (h, 0, 0)),
            _full((CP, HALF)),
            _full((CP, HALF)),
        ],
        out_specs=[
            pl.BlockSpec((1, 1, CP, D), lambda b, h: (b, h, 0, 0)),
            pl.BlockSpec((1, 1, CP, D), lambda b, h: (b, h, 0, 0)),
        ],
        out_shape=[
            jax.ShapeDtypeStruct((B, HK, CP, D), F32),
            jax.ShapeDtypeStruct((B, HK, CP, D), F32),
        ],
    )(kraw, v, Wck, Wcv, pe, COS_C, SIN_C)

    comp, bm = pl.pallas_call(
        _select_kernel,
        grid=(B, HK),
        in_specs=[
            pl.BlockSpec((1, L, G * D), lambda b, h: (b, 0, h)),
            pl.BlockSpec((1, 1, CP, D), lambda b, h: (b, h, 0, 0)),
            pl.BlockSpec((1, 1, CP, D), lambda b, h: (b, h, 0, 0)),
            _full((L, CP)),
            _full((CP, NB)),
            _full((L, NB)),
            _full((L, NB)),
        ],
        out_specs=[
            pl.BlockSpec((1, 1, G, L, D), lambda b, h: (b, h, 0, 0, 0)),
            pl.BlockSpec((1, 1, L, NB), lambda b, h: (b, h, 0, 0)),
        ],
        out_shape=[
            jax.ShapeDtypeStruct((B, HK, G, L, D), F32),
            jax.ShapeDtypeStruct((B, HK, L, NB), F32),
        ],
    )(q, ck, cv, CMASK, OVERLAP, FORCEDV, FUTURE)

    att = pl.pallas_call(
        _attn_kernel,
        grid=(B, HK, NQB),
        in_specs=[
            pl.BlockSpec((1, QB, G * D), lambda b, h, i: (b, i, h)),
            pl.BlockSpec((1, 1, L, D), lambda b, h, i: (b, h, 0, 0)),
            pl.BlockSpec((1, 1, L, D), lambda b, h, i: (b, h, 0, 0)),
            pl.BlockSpec((1, 1, G, QB, D), lambda b, h, i: (b, h, 0, i, 0)),
            pl.BlockSpec((1, 1, QB, NB), lambda b, h, i: (b, h, i, 0)),
            pl.BlockSpec((1, QB, 8), lambda b, h, i: (b, i, 0)),
            _full((NB, L)),
        ],
        out_specs=pl.BlockSpec((1, QB, G * D), lambda b, h, i: (b, i, h)),
        out_shape=jax.ShapeDtypeStruct((B, L, HQ * D), F32),
    )(q, krot, v, comp, bm, gate, EXPAND)

    out = pl.pallas_call(
        _outproj_kernel,
        grid=(N // RB,),
        in_specs=[
            pl.BlockSpec((RB, HQ * D), lambda i: (i, 0)),
            _full((HQ * D, H)),
        ],
        out_specs=pl.BlockSpec((RB, H), lambda i: (i, 0)),
        out_shape=jax.ShapeDtypeStruct((N, H), F32),
    )(att.reshape(N, HQ * D), Wo)

    return out
